# baseline (speedup 1.0000x reference)
import functools

import jax
import jax.numpy as jnp
from jax import lax
from jax.experimental import pallas as pl
from jax.experimental.pallas import tpu as pltpu
from jax.experimental.pallas import tpu_sc as plsc

S = 2048
D = 1024
H = 1024
E = 8
P = 16
K = 2
TM = 512
M2 = S * K + E * TM
NT = M2 // TM
SB = 1024


def _router_body(pids_ref, x_ref, wc_ref, pr_ref, comb_ref, w0_ref, w1_ref,
                 z_ref, p_ref, s_ref):
    i = pl.program_id(0)
    xb = x_ref[...]
    wc = wc_ref[...]
    logits = lax.dot_general(xb, wc, (((1,), (1,)), ((), ())),
                             preferred_element_type=jnp.float32)
    pid = pids_ref[0]
    bias = pr_ref[pl.ds(pid, 1), :]
    logits = logits + bias

    lane = lax.broadcasted_iota(jnp.int32, (SB, E), 1)
    m1 = jnp.max(logits, axis=1, keepdims=True)
    i1 = jnp.min(jnp.where(logits == m1, lane, E), axis=1, keepdims=True)
    masked = jnp.where(lane == i1, -jnp.inf, logits)
    m2 = jnp.max(masked, axis=1, keepdims=True)
    i2 = jnp.min(jnp.where(masked == m2, lane, E), axis=1, keepdims=True)
    b = jnp.exp(m2 - m1)
    wa = 1.0 / (1.0 + b)
    comb_ref[...] = jnp.where(
        lane == 0, i1.astype(jnp.float32),
        jnp.where(lane == 1, i2.astype(jnp.float32),
                  jnp.where(lane == 2, wa,
                            jnp.where(lane == 3, 1.0 - wa, 0.0))))
    w0_ref[...] = jnp.broadcast_to(wa, (SB, 16))
    w1_ref[...] = jnp.broadcast_to(1.0 - wa, (SB, 16))

    mx = jnp.max(logits, axis=1, keepdims=True)
    ex = jnp.exp(logits - mx)
    probs = ex / jnp.sum(ex, axis=1, keepdims=True)

    @pl.when(i == 0)
    def _():
        z_ref[0, 0] = 0.0
        p_ref[...] = jnp.zeros_like(p_ref)
        prm = pr_ref[...]
        pmx = jnp.max(prm, axis=1, keepdims=True)
        pex = jnp.exp(prm - pmx)
        nw = pex / jnp.sum(pex, axis=1, keepdims=True)
        sim = lax.dot_general(nw, nw, (((1,), (1,)), ((), ())),
                              preferred_element_type=jnp.float32)
        r = lax.broadcasted_iota(jnp.int32, (P, P), 0)
        c = lax.broadcasted_iota(jnp.int32, (P, P), 1)
        s_ref[0, 0] = jnp.sum(jnp.where(r == c, 0.0, sim)) / (P * P) * 0.1

    z_ref[0, 0] += jnp.sum(logits * logits)
    p_ref[...] += jnp.sum(probs, axis=0, keepdims=True)


def _router_tc(x2, pattern_ids, W_content, pattern_routing):
    nblk = S // SB
    comb, w0r, w1r, zsum, psum, spec = pl.pallas_call(
        _router_body,
        grid=(nblk,),
        in_specs=[
            pl.BlockSpec(memory_space=pltpu.SMEM),
            pl.BlockSpec((SB, D), lambda i: (i, 0)),
            pl.BlockSpec((E, D), lambda i: (0, 0)),
            pl.BlockSpec((P, E), lambda i: (0, 0)),
        ],
        out_specs=[
            pl.BlockSpec((SB, E), lambda i: (i, 0)),
            pl.BlockSpec((SB, 16), lambda i: (i, 0)),
            pl.BlockSpec((SB, 16), lambda i: (i, 0)),
            pl.BlockSpec(memory_space=pltpu.SMEM, block_shape=(1, 1),
                         index_map=lambda i: (0, 0)),
            pl.BlockSpec((1, E), lambda i: (0, 0)),
            pl.BlockSpec(memory_space=pltpu.SMEM, block_shape=(1, 1),
                         index_map=lambda i: (0, 0)),
        ],
        out_shape=[
            jax.ShapeDtypeStruct((S, E), jnp.float32),
            jax.ShapeDtypeStruct((S, 16), jnp.float32),
            jax.ShapeDtypeStruct((S, 16), jnp.float32),
            jax.ShapeDtypeStruct((1, 1), jnp.float32),
            jax.ShapeDtypeStruct((1, E), jnp.float32),
            jax.ShapeDtypeStruct((1, 1), jnp.float32),
        ],
    )(pattern_ids.astype(jnp.int32), x2, W_content, pattern_routing)
    return comb, w0r, w1r, zsum, psum, spec


def _metadata(comb):
    lane = jnp.arange(E, dtype=jnp.int32)[None, :]
    i1 = comb[:, 0].astype(jnp.int32)
    i2 = comb[:, 1].astype(jnp.int32)
    oh1 = (i1[:, None] == lane).astype(jnp.float32)
    oh2 = (i2[:, None] == lane).astype(jnp.float32)
    oh = jnp.stack([oh1, oh2], axis=1).reshape(S * K, E)
    NB, BL = 32, (S * K) // 32
    ohb = oh.reshape(NB, BL, E)
    trilb = jnp.tril(jnp.ones((BL, BL), jnp.float32))
    within = jnp.einsum("rc,bce->bre", trilb, ohb)
    btot = ohb.sum(axis=1)
    trilx = jnp.tril(jnp.ones((NB, NB), jnp.float32), k=-1)
    bpref = jnp.einsum("xb,be->xe", trilx, btot)
    csum = (bpref[:, None, :] + within).reshape(S * K, E)
    counts = csum[-1].astype(jnp.int32)
    pcounts = ((counts + TM - 1) // TM) * TM
    poffs = jnp.concatenate([jnp.zeros((1,), jnp.int32),
                             jnp.cumsum(pcounts)[:-1]])
    total = jnp.sum(pcounts)
    slot_p = jnp.sum(oh * (poffs[None, :].astype(jnp.float32) + csum - 1.0),
                     axis=1).astype(jnp.int32)
    slot0 = slot_p[0::2]
    slot1 = slot_p[1::2]
    tile_start = jnp.arange(NT, dtype=jnp.int32) * TM
    tile_expert = jnp.clip(
        jnp.searchsorted(poffs, tile_start, side="right").astype(jnp.int32) - 1,
        0, E - 1)
    tile_valid = (tile_start < total).astype(jnp.int32)
    return slot0, slot1, tile_expert, tile_valid


def _expert_body(te_ref, tv_ref, xs_ref, w1_ref, w3_ref, w2_ref, o_ref):
    i = pl.program_id(0)

    @pl.when(tv_ref[i] == 1)
    def _():
        xb = xs_ref[...]
        w1b = w1_ref[0]
        w3b = w3_ref[0]
        w2b = w2_ref[0]
        h = lax.dot_general(xb, w1b, (((1,), (1,)), ((), ())),
                            precision=lax.Precision.DEFAULT,
                            preferred_element_type=jnp.float32)
        g = lax.dot_general(xb, w3b, (((1,), (1,)), ((), ())),
                            precision=lax.Precision.DEFAULT,
                            preferred_element_type=jnp.float32)
        g = g * (1.0 / (1.0 + jnp.exp(-g)))
        hg = h * g
        o_ref[...] = lax.dot_general(hg, w2b, (((1,), (1,)), ((), ())),
                                     precision=lax.Precision.DEFAULT,
                                     preferred_element_type=jnp.float32)


def _experts_tc(xs, tile_expert, tile_valid, w1b, w2b, w3b):
    grid_spec = pltpu.PrefetchScalarGridSpec(
        num_scalar_prefetch=2,
        grid=(NT,),
        in_specs=[
            pl.BlockSpec((TM, D), lambda i, te, tv: (i, 0)),
            pl.BlockSpec((1, H, D), lambda i, te, tv: (te[i], 0, 0)),
            pl.BlockSpec((1, H, D), lambda i, te, tv: (te[i], 0, 0)),
            pl.BlockSpec((1, D, H), lambda i, te, tv: (te[i], 0, 0)),
        ],
        out_specs=pl.BlockSpec((TM, D), lambda i, te, tv: (i, 0)),
    )
    return pl.pallas_call(
        _expert_body,
        grid_spec=grid_spec,
        out_shape=jax.ShapeDtypeStruct((M2, D), jnp.float32),
    )(tile_expert, tile_valid, xs, w1b, w3b, w2b)


def _sc_workers():
    info = plsc.get_sparse_core_info()
    return info.num_cores, info.num_subcores


def _dispatch_sc(x2, slot0, slot1):
    NC, NS = _sc_workers()
    NW = NC * NS
    per_w = S // NW
    CH = 16
    nch = per_w // CH
    NRB = 3
    mesh = plsc.VectorSubcoreMesh(core_axis_name="c", subcore_axis_name="s")

    @functools.partial(
        pl.kernel, mesh=mesh,
        out_type=jax.ShapeDtypeStruct((M2, D), jnp.float32),
        scratch_types=(
            [pltpu.VMEM((CH,), jnp.int32) for _ in range(2 * nch)]
            + [pltpu.VMEM((CH, D), jnp.float32) for _ in range(NRB)]
            + [pltpu.SemaphoreType.DMA, pltpu.SemaphoreType.DMA]
        ),
    )
    def dk(s0_hbm, s1_hbm, x_hbm, out_hbm, *scr):
        idx = scr[:2 * nch]
        rbufs = scr[2 * nch:2 * nch + NRB]
        rsem, ssem = scr[2 * nch + NRB:]
        wid = lax.axis_index("s") * NC + lax.axis_index("c")
        base = wid * per_w
        reads = [None] * nch
        scat = [None] * (2 * nch)
        for c in range(2):
            pltpu.sync_copy(s0_hbm.at[pl.ds(base + c * CH, CH)], idx[2 * c])
            pltpu.sync_copy(s1_hbm.at[pl.ds(base + c * CH, CH)], idx[2 * c + 1])
            reads[c] = pltpu.async_copy(x_hbm.at[pl.ds(base + c * CH, CH)],
                                        rbufs[c], rsem)
        for c in range(nch):
            if c + 2 < nch:
                if c >= 1:
                    scat[2 * (c - 1)].wait()
                    scat[2 * (c - 1) + 1].wait()
                pltpu.sync_copy(s0_hbm.at[pl.ds(base + (c + 2) * CH, CH)],
                                idx[2 * (c + 2)])
                pltpu.sync_copy(s1_hbm.at[pl.ds(base + (c + 2) * CH, CH)],
                                idx[2 * (c + 2) + 1])
                reads[c + 2] = pltpu.async_copy(
                    x_hbm.at[pl.ds(base + (c + 2) * CH, CH)],
                    rbufs[(c + 2) % NRB], rsem)
            reads[c].wait()
            scat[2 * c] = pltpu.async_copy(
                rbufs[c % NRB], out_hbm.at[idx[2 * c]], ssem)
            scat[2 * c + 1] = pltpu.async_copy(
                rbufs[c % NRB], out_hbm.at[idx[2 * c + 1]], ssem)
        for c in range(max(0, nch - 3), nch):
            scat[2 * c].wait()
            scat[2 * c + 1].wait()

    return dk(slot0, slot1, x2)


def _combine_sc(out_s, slot0, slot1, w0r, w1r):
    NC, NS = _sc_workers()
    NW = NC * NS
    per_w = S // NW
    CH = 16
    nch = per_w // CH
    NBUF = 3
    mesh = plsc.VectorSubcoreMesh(core_axis_name="c", subcore_axis_name="s")

    @functools.partial(
        pl.kernel, mesh=mesh,
        out_type=jax.ShapeDtypeStruct((S, D), jnp.float32),
        scratch_types=[
            pltpu.VMEM((per_w,), jnp.int32),
            pltpu.VMEM((per_w,), jnp.int32),
            pltpu.VMEM((per_w, 16), jnp.float32),
            pltpu.VMEM((per_w, 16), jnp.float32),
            pltpu.VMEM((CH, D), jnp.float32),
            pltpu.VMEM((CH, D), jnp.float32),
            pltpu.VMEM((CH, D), jnp.float32),
            pltpu.VMEM((CH, D), jnp.float32),
            pltpu.VMEM((CH, D), jnp.float32),
            pltpu.VMEM((CH, D), jnp.float32),
            pltpu.SemaphoreType.DMA,
            pltpu.SemaphoreType.DMA,
        ],
    )
    def ck(s0_hbm, s1_hbm, w0_hbm, w1_hbm, rows_hbm, y_hbm, i0_v, i1_v,
           w0_v, w1_v, a0_v, b0_v, a1_v, b1_v, a2_v, b2_v, gsem, wsem):
        wid = lax.axis_index("s") * NC + lax.axis_index("c")
        base = wid * per_w
        abufs = (a0_v, a1_v, a2_v)
        bbufs = (b0_v, b1_v, b2_v)
        pltpu.sync_copy(s0_hbm.at[pl.ds(base, per_w)], i0_v)
        pltpu.sync_copy(s1_hbm.at[pl.ds(base, per_w)], i1_v)
        pltpu.sync_copy(w0_hbm.at[pl.ds(base, per_w)], w0_v)
        pltpu.sync_copy(w1_hbm.at[pl.ds(base, per_w)], w1_v)
        ga = [None] * nch
        gb = [None] * nch
        writes = [None] * nch
        for c in range(2):
            ga[c] = pltpu.async_copy(rows_hbm.at[i0_v.at[pl.ds(c * CH, CH)]],
                                     abufs[c], gsem)
            gb[c] = pltpu.async_copy(rows_hbm.at[i1_v.at[pl.ds(c * CH, CH)]],
                                     bbufs[c], gsem)
        for c in range(nch):
            if c + 2 < nch:
                if c >= 1:
                    writes[c - 1].wait()
                ga[c + 2] = pltpu.async_copy(
                    rows_hbm.at[i0_v.at[pl.ds((c + 2) * CH, CH)]],
                    abufs[(c + 2) % NBUF], gsem)
                gb[c + 2] = pltpu.async_copy(
                    rows_hbm.at[i1_v.at[pl.ds((c + 2) * CH, CH)]],
                    bbufs[(c + 2) % NBUF], gsem)
            ga[c].wait()
            gb[c].wait()
            a_v = abufs[c % NBUF]
            b_v = bbufs[c % NBUF]

            def row(r, _, a_v=a_v, b_v=b_v, c=c):
                w0 = w0_v[c * CH + r, 0:16]
                w1 = w1_v[c * CH + r, 0:16]
                for k in range(D // 16):
                    sl = slice(k * 16, (k + 1) * 16)
                    a_v[r, sl] = a_v[r, sl] * w0 + b_v[r, sl] * w1
                return 0

            lax.fori_loop(0, CH, row, 0)
            writes[c] = pltpu.async_copy(
                a_v, y_hbm.at[pl.ds(base + c * CH, CH)], wsem)
        writes[nch - 3].wait()
        writes[nch - 2].wait()
        writes[nch - 1].wait()

    return ck(slot0, slot1, w0r, w1r, out_s)


def kernel(x, pattern_ids, W_content, pattern_routing, w1, w2, w3):
    x2 = x.reshape(S, D)
    comb, w0r, w1r, zsum, psum, spec = _router_tc(x2, pattern_ids, W_content,
                                                  pattern_routing)
    slot0, slot1, tile_expert, tile_valid = _metadata(comb)
    xs = _dispatch_sc(x2, slot0, slot1)
    out_s = _experts_tc(xs, tile_expert, tile_valid, w1, w2, w3)
    y = _combine_sc(out_s, slot0, slot1, w0r, w1r)

    router_z_loss = zsum[0, 0] / (S * E) * 0.001
    pm = psum[0] / S
    balance_loss = jnp.mean(jnp.square(pm - 1.0 / E))
    specialization_loss = spec[0, 0]
    return (y.reshape(1, S, D), router_z_loss, balance_loss,
            specialization_loss)


# speedup vs baseline: 1.0065x; 1.0065x over previous
import functools

import jax
import jax.numpy as jnp
from jax import lax
from jax.experimental import pallas as pl
from jax.experimental.pallas import tpu as pltpu
from jax.experimental.pallas import tpu_sc as plsc

S = 2048
D = 1024
H = 1024
E = 8
P = 16
K = 2
TM = 512
M2 = S * K + E * TM
NT = M2 // TM
SB = 1024


def _router_body(pids_ref, x_ref, wc_ref, pr_ref, comb_ref, w0_ref, w1_ref,
                 z_ref, p_ref, s_ref):
    i = pl.program_id(0)
    xb = x_ref[...]
    wc = wc_ref[...]
    logits = lax.dot_general(xb, wc, (((1,), (1,)), ((), ())),
                             preferred_element_type=jnp.float32)
    pid = pids_ref[0]
    bias = pr_ref[pl.ds(pid, 1), :]
    logits = logits + bias

    lane = lax.broadcasted_iota(jnp.int32, (SB, E), 1)
    m1 = jnp.max(logits, axis=1, keepdims=True)
    i1 = jnp.min(jnp.where(logits == m1, lane, E), axis=1, keepdims=True)
    masked = jnp.where(lane == i1, -jnp.inf, logits)
    m2 = jnp.max(masked, axis=1, keepdims=True)
    i2 = jnp.min(jnp.where(masked == m2, lane, E), axis=1, keepdims=True)
    b = jnp.exp(m2 - m1)
    wa = 1.0 / (1.0 + b)
    comb_ref[...] = jnp.where(
        lane == 0, i1.astype(jnp.float32),
        jnp.where(lane == 1, i2.astype(jnp.float32),
                  jnp.where(lane == 2, wa,
                            jnp.where(lane == 3, 1.0 - wa, 0.0))))
    w0_ref[...] = jnp.broadcast_to(wa, (SB, 16))
    w1_ref[...] = jnp.broadcast_to(1.0 - wa, (SB, 16))

    mx = jnp.max(logits, axis=1, keepdims=True)
    ex = jnp.exp(logits - mx)
    probs = ex / jnp.sum(ex, axis=1, keepdims=True)

    @pl.when(i == 0)
    def _():
        z_ref[0, 0] = 0.0
        p_ref[...] = jnp.zeros_like(p_ref)
        prm = pr_ref[...]
        pmx = jnp.max(prm, axis=1, keepdims=True)
        pex = jnp.exp(prm - pmx)
        nw = pex / jnp.sum(pex, axis=1, keepdims=True)
        sim = lax.dot_general(nw, nw, (((1,), (1,)), ((), ())),
                              preferred_element_type=jnp.float32)
        r = lax.broadcasted_iota(jnp.int32, (P, P), 0)
        c = lax.broadcasted_iota(jnp.int32, (P, P), 1)
        s_ref[0, 0] = jnp.sum(jnp.where(r == c, 0.0, sim)) / (P * P) * 0.1

    z_ref[0, 0] += jnp.sum(logits * logits)
    p_ref[...] += jnp.sum(probs, axis=0, keepdims=True)


def _router_tc(x2, pattern_ids, W_content, pattern_routing):
    nblk = S // SB
    comb, w0r, w1r, zsum, psum, spec = pl.pallas_call(
        _router_body,
        grid=(nblk,),
        in_specs=[
            pl.BlockSpec(memory_space=pltpu.SMEM),
            pl.BlockSpec((SB, D), lambda i: (i, 0)),
            pl.BlockSpec((E, D), lambda i: (0, 0)),
            pl.BlockSpec((P, E), lambda i: (0, 0)),
        ],
        out_specs=[
            pl.BlockSpec((SB, E), lambda i: (i, 0)),
            pl.BlockSpec((SB, 16), lambda i: (i, 0)),
            pl.BlockSpec((SB, 16), lambda i: (i, 0)),
            pl.BlockSpec(memory_space=pltpu.SMEM, block_shape=(1, 1),
                         index_map=lambda i: (0, 0)),
            pl.BlockSpec((1, E), lambda i: (0, 0)),
            pl.BlockSpec(memory_space=pltpu.SMEM, block_shape=(1, 1),
                         index_map=lambda i: (0, 0)),
        ],
        out_shape=[
            jax.ShapeDtypeStruct((S, E), jnp.float32),
            jax.ShapeDtypeStruct((S, 16), jnp.float32),
            jax.ShapeDtypeStruct((S, 16), jnp.float32),
            jax.ShapeDtypeStruct((1, 1), jnp.float32),
            jax.ShapeDtypeStruct((1, E), jnp.float32),
            jax.ShapeDtypeStruct((1, 1), jnp.float32),
        ],
    )(pattern_ids.astype(jnp.int32), x2, W_content, pattern_routing)
    return comb, w0r, w1r, zsum, psum, spec


def _metadata(comb):
    lane = jnp.arange(E, dtype=jnp.int32)[None, :]
    i1 = comb[:, 0].astype(jnp.int32)
    i2 = comb[:, 1].astype(jnp.int32)
    oh1 = (i1[:, None] == lane).astype(jnp.float32)
    oh2 = (i2[:, None] == lane).astype(jnp.float32)
    oh = jnp.stack([oh1, oh2], axis=1).reshape(S * K, E)
    NB, BL = 32, (S * K) // 32
    ohb = oh.reshape(NB, BL, E)
    trilb = jnp.tril(jnp.ones((BL, BL), jnp.float32))
    within = jnp.einsum("rc,bce->bre", trilb, ohb)
    btot = ohb.sum(axis=1)
    trilx = jnp.tril(jnp.ones((NB, NB), jnp.float32), k=-1)
    bpref = jnp.einsum("xb,be->xe", trilx, btot)
    csum = (bpref[:, None, :] + within).reshape(S * K, E)
    counts = csum[-1].astype(jnp.int32)
    pcounts = ((counts + TM - 1) // TM) * TM
    poffs = jnp.concatenate([jnp.zeros((1,), jnp.int32),
                             jnp.cumsum(pcounts)[:-1]])
    total = jnp.sum(pcounts)
    slot_p = jnp.sum(oh * (poffs[None, :].astype(jnp.float32) + csum - 1.0),
                     axis=1).astype(jnp.int32)
    slot0 = slot_p[0::2]
    slot1 = slot_p[1::2]
    tile_start = jnp.arange(NT, dtype=jnp.int32) * TM
    tile_expert = jnp.clip(
        jnp.searchsorted(poffs, tile_start, side="right").astype(jnp.int32) - 1,
        0, E - 1)
    tile_valid = (tile_start < total).astype(jnp.int32)
    return slot0, slot1, tile_expert, tile_valid


def _expert_body(te_ref, tv_ref, xs_ref, w1_ref, w3_ref, w2_ref, o_ref):
    i = pl.program_id(0)

    @pl.when(tv_ref[i] == 1)
    def _():
        xb = xs_ref[...]
        w1b = w1_ref[0]
        w3b = w3_ref[0]
        w2b = w2_ref[0]
        h = lax.dot_general(xb, w1b, (((1,), (1,)), ((), ())),
                            precision=lax.Precision.DEFAULT,
                            preferred_element_type=jnp.float32)
        g = lax.dot_general(xb, w3b, (((1,), (1,)), ((), ())),
                            precision=lax.Precision.DEFAULT,
                            preferred_element_type=jnp.float32)
        g = g * (1.0 / (1.0 + jnp.exp(-g)))
        hg = h * g
        o_ref[...] = lax.dot_general(hg, w2b, (((1,), (1,)), ((), ())),
                                     precision=lax.Precision.DEFAULT,
                                     preferred_element_type=jnp.float32)


def _experts_tc(xs, tile_expert, tile_valid, w1b, w2b, w3b):
    grid_spec = pltpu.PrefetchScalarGridSpec(
        num_scalar_prefetch=2,
        grid=(NT,),
        in_specs=[
            pl.BlockSpec((TM, D), lambda i, te, tv: (i, 0)),
            pl.BlockSpec((1, H, D), lambda i, te, tv: (te[i], 0, 0)),
            pl.BlockSpec((1, H, D), lambda i, te, tv: (te[i], 0, 0)),
            pl.BlockSpec((1, D, H), lambda i, te, tv: (te[i], 0, 0)),
        ],
        out_specs=pl.BlockSpec((TM, D), lambda i, te, tv: (i, 0)),
    )
    return pl.pallas_call(
        _expert_body,
        grid_spec=grid_spec,
        out_shape=jax.ShapeDtypeStruct((M2, D), jnp.float32),
    )(tile_expert, tile_valid, xs, w1b, w3b, w2b)


def _sc_workers():
    info = plsc.get_sparse_core_info()
    return info.num_cores, info.num_subcores


def _dispatch_sc(x2, slot0, slot1):
    NC, NS = _sc_workers()
    NW = NC * NS
    per_w = S // NW
    CH = 32
    nch = per_w // CH
    mesh = plsc.VectorSubcoreMesh(core_axis_name="c", subcore_axis_name="s")

    @functools.partial(
        pl.kernel, mesh=mesh,
        out_type=jax.ShapeDtypeStruct((M2, D), jnp.float32),
        scratch_types=[
            pltpu.VMEM((CH,), jnp.int32),
            pltpu.VMEM((CH,), jnp.int32),
            pltpu.VMEM((CH,), jnp.int32),
            pltpu.VMEM((CH,), jnp.int32),
            pltpu.VMEM((CH, D), jnp.float32),
            pltpu.VMEM((CH, D), jnp.float32),
            pltpu.SemaphoreType.DMA,
            pltpu.SemaphoreType.DMA,
        ],
    )
    def dk(s0_hbm, s1_hbm, x_hbm, out_hbm, i0a_v, i1a_v, i0b_v, i1b_v,
           r0_v, r1_v, rsem, ssem):
        wid = lax.axis_index("s") * NC + lax.axis_index("c")
        base = wid * per_w
        ibufs = ((i0a_v, i1a_v), (i0b_v, i1b_v))
        rbufs = (r0_v, r1_v)
        reads = [None] * nch
        scat = [None] * (2 * nch)
        pltpu.sync_copy(s0_hbm.at[pl.ds(base, CH)], i0a_v)
        pltpu.sync_copy(s1_hbm.at[pl.ds(base, CH)], i1a_v)
        reads[0] = pltpu.async_copy(x_hbm.at[pl.ds(base, CH)], r0_v, rsem)
        for c in range(nch):
            if c + 1 < nch:
                pltpu.sync_copy(s0_hbm.at[pl.ds(base + (c + 1) * CH, CH)],
                                ibufs[(c + 1) % 2][0])
                pltpu.sync_copy(s1_hbm.at[pl.ds(base + (c + 1) * CH, CH)],
                                ibufs[(c + 1) % 2][1])
                reads[c + 1] = pltpu.async_copy(
                    x_hbm.at[pl.ds(base + (c + 1) * CH, CH)],
                    rbufs[(c + 1) % 2], rsem)
            reads[c].wait()
            scat[2 * c] = pltpu.async_copy(
                rbufs[c % 2], out_hbm.at[ibufs[c % 2][0]], ssem)
            scat[2 * c + 1] = pltpu.async_copy(
                rbufs[c % 2], out_hbm.at[ibufs[c % 2][1]], ssem)
        for s in scat:
            s.wait()

    return dk(slot0, slot1, x2)


def _combine_sc(out_s, slot0, slot1, w0r, w1r):
    NC, NS = _sc_workers()
    NW = NC * NS
    per_w = S // NW
    CH = 16
    nch = per_w // CH
    NBUF = 3
    mesh = plsc.VectorSubcoreMesh(core_axis_name="c", subcore_axis_name="s")

    @functools.partial(
        pl.kernel, mesh=mesh,
        out_type=jax.ShapeDtypeStruct((S, D), jnp.float32),
        scratch_types=[
            pltpu.VMEM((per_w,), jnp.int32),
            pltpu.VMEM((per_w,), jnp.int32),
            pltpu.VMEM((per_w, 16), jnp.float32),
            pltpu.VMEM((per_w, 16), jnp.float32),
            pltpu.VMEM((CH, D), jnp.float32),
            pltpu.VMEM((CH, D), jnp.float32),
            pltpu.VMEM((CH, D), jnp.float32),
            pltpu.VMEM((CH, D), jnp.float32),
            pltpu.VMEM((CH, D), jnp.float32),
            pltpu.VMEM((CH, D), jnp.float32),
            pltpu.SemaphoreType.DMA,
            pltpu.SemaphoreType.DMA,
        ],
    )
    def ck(s0_hbm, s1_hbm, w0_hbm, w1_hbm, rows_hbm, y_hbm, i0_v, i1_v,
           w0_v, w1_v, a0_v, b0_v, a1_v, b1_v, a2_v, b2_v, gsem, wsem):
        wid = lax.axis_index("s") * NC + lax.axis_index("c")
        base = wid * per_w
        abufs = (a0_v, a1_v, a2_v)
        bbufs = (b0_v, b1_v, b2_v)
        pltpu.sync_copy(s0_hbm.at[pl.ds(base, per_w)], i0_v)
        pltpu.sync_copy(s1_hbm.at[pl.ds(base, per_w)], i1_v)
        pltpu.sync_copy(w0_hbm.at[pl.ds(base, per_w)], w0_v)
        pltpu.sync_copy(w1_hbm.at[pl.ds(base, per_w)], w1_v)
        ga = [None] * nch
        gb = [None] * nch
        writes = [None] * nch
        for c in range(2):
            ga[c] = pltpu.async_copy(rows_hbm.at[i0_v.at[pl.ds(c * CH, CH)]],
                                     abufs[c], gsem)
            gb[c] = pltpu.async_copy(rows_hbm.at[i1_v.at[pl.ds(c * CH, CH)]],
                                     bbufs[c], gsem)
        for c in range(nch):
            if c + 2 < nch:
                if c >= 1:
                    writes[c - 1].wait()
                ga[c + 2] = pltpu.async_copy(
                    rows_hbm.at[i0_v.at[pl.ds((c + 2) * CH, CH)]],
                    abufs[(c + 2) % NBUF], gsem)
                gb[c + 2] = pltpu.async_copy(
                    rows_hbm.at[i1_v.at[pl.ds((c + 2) * CH, CH)]],
                    bbufs[(c + 2) % NBUF], gsem)
            ga[c].wait()
            gb[c].wait()
            a_v = abufs[c % NBUF]
            b_v = bbufs[c % NBUF]

            def row(r, _, a_v=a_v, b_v=b_v, c=c):
                w0 = w0_v[c * CH + r, 0:16]
                w1 = w1_v[c * CH + r, 0:16]
                for k in range(D // 16):
                    sl = slice(k * 16, (k + 1) * 16)
                    a_v[r, sl] = a_v[r, sl] * w0 + b_v[r, sl] * w1
                return 0

            lax.fori_loop(0, CH, row, 0)
            writes[c] = pltpu.async_copy(
                a_v, y_hbm.at[pl.ds(base + c * CH, CH)], wsem)
        writes[nch - 3].wait()
        writes[nch - 2].wait()
        writes[nch - 1].wait()

    return ck(slot0, slot1, w0r, w1r, out_s)


def kernel(x, pattern_ids, W_content, pattern_routing, w1, w2, w3):
    x2 = x.reshape(S, D)
    comb, w0r, w1r, zsum, psum, spec = _router_tc(x2, pattern_ids, W_content,
                                                  pattern_routing)
    slot0, slot1, tile_expert, tile_valid = _metadata(comb)
    xs = _dispatch_sc(x2, slot0, slot1)
    out_s = _experts_tc(xs, tile_expert, tile_valid, w1, w2, w3)
    y = _combine_sc(out_s, slot0, slot1, w0r, w1r)

    router_z_loss = zsum[0, 0] / (S * E) * 0.001
    pm = psum[0] / S
    balance_loss = jnp.mean(jnp.square(pm - 1.0 / E))
    specialization_loss = spec[0, 0]
    return (y.reshape(1, S, D), router_z_loss, balance_loss,
            specialization_loss)
